# baseline (speedup 1.0000x reference)
import functools

import jax
import jax.numpy as jnp
from jax import lax
from jax.experimental import pallas as pl
from jax.experimental.pallas import tpu as pltpu
from jax.experimental.pallas import tpu_sc as plsc

_NC, _NS, _NL = 2, 16, 16

_TOPK = 2
_BLK = 576
_IB = 1024


def _ffn_body(bemap_ref, nbc_ref, valid_ref, xs_ref, fc1_ref, fc3_ref,
              fc2_ref, out_ref, acc_ref, *, n_it):
    nb = pl.program_id(0)
    it = pl.program_id(1)

    @pl.when(valid_ref[nb] == 1)
    def _():
        @pl.when(it == 0)
        def _():
            acc_ref[...] = jnp.zeros_like(acc_ref)

        xs = xs_ref[...].astype(jnp.bfloat16)
        h1 = jnp.dot(xs, fc1_ref[0].astype(jnp.bfloat16),
                     preferred_element_type=jnp.float32)
        h3 = jnp.dot(xs, fc3_ref[0].astype(jnp.bfloat16),
                     preferred_element_type=jnp.float32)
        act = h1 * jax.nn.sigmoid(h1) * h3
        acc_ref[...] += jnp.dot(act.astype(jnp.bfloat16),
                                fc2_ref[0].astype(jnp.bfloat16),
                                preferred_element_type=jnp.float32)

        @pl.when(it == n_it - 1)
        def _():
            out_ref[...] = acc_ref[...]


def _grouped_ffn(xs, fc1, fc3, fc2, bemap, nbclamp, valid):
    p, h = xs.shape
    _, _, inter = fc1.shape
    n_nb = p // _BLK
    n_it = inter // _IB

    def wmap(nb, it, bm, nc, vl):
        return (bm[nb], jnp.where(vl[nb] == 1, it, n_it - 1))

    def map13(nb, it, bm, nc, vl):
        be_i, it_i = wmap(nb, it, bm, nc, vl)
        return (be_i, 0, it_i)

    def map2(nb, it, bm, nc, vl):
        be_i, it_i = wmap(nb, it, bm, nc, vl)
        return (be_i, it_i, 0)

    return pl.pallas_call(
        functools.partial(_ffn_body, n_it=n_it),
        grid_spec=pltpu.PrefetchScalarGridSpec(
            num_scalar_prefetch=3,
            grid=(n_nb, n_it),
            in_specs=[
                pl.BlockSpec((_BLK, h), lambda nb, it, bm, nc, vl: (nc[nb], 0)),
                pl.BlockSpec((1, h, _IB), map13),
                pl.BlockSpec((1, h, _IB), map13),
                pl.BlockSpec((1, _IB, h), map2),
            ],
            out_specs=pl.BlockSpec((_BLK, h), lambda nb, it, bm, nc, vl: (nc[nb], 0)),
            scratch_shapes=[pltpu.VMEM((_BLK, h), jnp.float32)],
        ),
        out_shape=jax.ShapeDtypeStruct((p, h), jnp.float32),
        compiler_params=pltpu.CompilerParams(
            dimension_semantics=("arbitrary", "arbitrary"),
        ),
    )(bemap, nbclamp, valid, xs, fc1, fc3, fc2)


def _vgather(vec, idx):
    return lax.gather(
        vec, idx[:, None],
        lax.GatherDimensionNumbers(offset_dims=(), collapsed_slice_dims=(0,),
                                   start_index_map=(0,)),
        (1,), mode=lax.GatherScatterMode.PROMISE_IN_BOUNDS)


def _vsplat(vec, i):
    return _vgather(vec, jnp.full((_NL,), i, dtype=jnp.int32))


def _scatter_sc(x, pos1, pos2, n, h, ptotal):
    nw = _NC * _NS
    tpw = n // nw
    mesh = plsc.VectorSubcoreMesh(core_axis_name="c", subcore_axis_name="s")

    @functools.partial(
        pl.kernel, mesh=mesh,
        out_type=jax.ShapeDtypeStruct((ptotal, h), jnp.float32),
        scratch_types=[
            pltpu.VMEM((tpw, h), jnp.float32),
            pltpu.VMEM((2, tpw), jnp.int32),
            pltpu.SemaphoreType.DMA,
        ],
    )
    def body(x_hbm, pos1_hbm, pos2_hbm, xs_hbm, xrows_v, idx2_v, sem):
        wid = lax.axis_index("s") * _NC + lax.axis_index("c")
        base = pl.ds(wid * tpw, tpw)
        pltpu.sync_copy(pos1_hbm.at[base], idx2_v.at[0])
        pltpu.sync_copy(pos2_hbm.at[base], idx2_v.at[1])
        pltpu.sync_copy(x_hbm.at[base], xrows_v)
        pltpu.async_copy(xrows_v, xs_hbm.at[idx2_v.at[0]], sem).wait()
        pltpu.async_copy(xrows_v, xs_hbm.at[idx2_v.at[1]], sem).wait()

    return body(x, pos1, pos2)


def _combine_sc(ys, posr, wr, n, h):
    nw = _NC * _NS
    tpw = n // nw
    nch = tpw // _NL
    mesh = plsc.VectorSubcoreMesh(core_axis_name="c", subcore_axis_name="s")

    @functools.partial(
        pl.kernel, mesh=mesh,
        out_type=jax.ShapeDtypeStruct((n, h), jnp.float32),
        scratch_types=[
            pltpu.VMEM((nch, 2 * _NL), jnp.int32),
            pltpu.VMEM((nch * 2 * _NL + _NL,), jnp.float32),
            pltpu.VMEM((2 * _NL, h), jnp.float32),
            pltpu.VMEM((_NL, h), jnp.float32),
            pltpu.SemaphoreType.DMA,
        ],
    )
    def body(ys_hbm, pos_hbm, w_hbm, out_hbm, idx_v, w_v, rows_v, out_v, sem):
        wid = lax.axis_index("s") * _NC + lax.axis_index("c")
        pltpu.sync_copy(pos_hbm.at[pl.ds(wid * nch, nch)], idx_v)
        pltpu.sync_copy(w_hbm.at[pl.ds(wid * nch * 2 * _NL, nch * 2 * _NL)],
                        w_v.at[pl.ds(0, nch * 2 * _NL)])
        for c in range(nch):
            pltpu.async_copy(ys_hbm.at[idx_v.at[c]], rows_v, sem).wait()

            def tok(t, _):
                wpair = w_v[pl.ds(c * 2 * _NL + 2 * t, _NL)]
                w1s = jnp.full((_NL,), wpair[0])
                w2s = jnp.full((_NL,), wpair[1])
                for sl in range(h // _NL):
                    d = pl.ds(sl * _NL, _NL)
                    out_v[t, d] = (w1s * rows_v[2 * t, d]
                                   + w2s * rows_v[2 * t + 1, d])
                return 0

            lax.fori_loop(0, _NL, tok, 0)
            pltpu.sync_copy(out_v, out_hbm.at[pl.ds(wid * tpw + c * _NL, _NL)])

    return body(ys, posr, wr)


def kernel(x, router_probs, fc1, fc2, fc3):
    n, h = x.shape
    e = fc1.shape[0]
    nk = n * _TOPK

    eids = jnp.arange(e, dtype=jnp.int32)
    a1 = jnp.argmax(router_probs, axis=1).astype(jnp.int32)
    m1 = jnp.max(router_probs, axis=1)
    rp2 = jnp.where(eids[None, :] == a1[:, None], -jnp.inf, router_probs)
    a2 = jnp.argmax(rp2, axis=1).astype(jnp.int32)
    m2 = jnp.max(rp2, axis=1)
    s = m1 + m2
    topk_probs = jnp.stack([m1 / s, m2 / s], axis=1)
    flat_e = jnp.stack([a1, a2], axis=1).reshape(-1)

    oh = (flat_e[:, None] == jnp.arange(e, dtype=jnp.int32)[None, :]).astype(jnp.int32)
    csum = jnp.cumsum(oh, axis=0)
    counts = csum[-1]
    rank = jnp.sum(csum * oh, axis=1) - 1

    padded = ((counts + _BLK - 1) // _BLK) * _BLK
    pstart = jnp.concatenate(
        [jnp.zeros((1,), jnp.int32), jnp.cumsum(padded)[:-1].astype(jnp.int32)])
    pos = pstart[flat_e] + rank

    ptotal = -(-(nk + e * _BLK) // _BLK) * _BLK
    n_nb = ptotal // _BLK

    ends = jnp.cumsum(padded).astype(jnp.int32)
    blk_starts = jnp.arange(n_nb, dtype=jnp.int32) * _BLK
    be = jnp.sum(blk_starts[:, None] >= ends[None, :], axis=1).astype(jnp.int32)
    valid = (be < e).astype(jnp.int32)
    last_valid = jnp.maximum(jnp.sum(valid) - 1, 0)
    nbclamp = jnp.minimum(jnp.arange(n_nb, dtype=jnp.int32), last_valid)
    bemap = jnp.minimum(be, be[last_valid])

    pos2d = pos.reshape(n, _TOPK)
    xs = _scatter_sc(x, pos2d[:, 0], pos2d[:, 1], n, h, ptotal)
    ys = _grouped_ffn(xs, fc1, fc3, fc2, bemap, nbclamp, valid)
    posr = pos.reshape(nk // (2 * _NL), 2 * _NL)
    wflat = topk_probs.reshape(-1)
    return _combine_sc(ys, posr, wflat, n, h)


# speedup vs baseline: 1.0236x; 1.0236x over previous
import functools

import jax
import jax.numpy as jnp
from jax import lax
from jax.experimental import pallas as pl
from jax.experimental.pallas import tpu as pltpu
from jax.experimental.pallas import tpu_sc as plsc

_NC, _NS, _NL = 2, 16, 16

_TOPK = 2
_BLK = 576
_IB = 1024


def _ffn_body(bemap_ref, nbc_ref, valid_ref, xs_ref, fc1_ref, fc3_ref,
              fc2_ref, out_ref, acc_ref, *, n_it):
    nb = pl.program_id(0)
    it = pl.program_id(1)

    @pl.when(valid_ref[nb] == 1)
    def _():
        @pl.when(it == 0)
        def _():
            acc_ref[...] = jnp.zeros_like(acc_ref)

        xs = xs_ref[...].astype(jnp.bfloat16)
        h1 = jnp.dot(xs, fc1_ref[0].astype(jnp.bfloat16),
                     preferred_element_type=jnp.float32)
        h3 = jnp.dot(xs, fc3_ref[0].astype(jnp.bfloat16),
                     preferred_element_type=jnp.float32)
        act = h1 * jax.nn.sigmoid(h1) * h3
        acc_ref[...] += jnp.dot(act.astype(jnp.bfloat16),
                                fc2_ref[0].astype(jnp.bfloat16),
                                preferred_element_type=jnp.float32)

        @pl.when(it == n_it - 1)
        def _():
            out_ref[...] = acc_ref[...]


def _grouped_ffn(xs, fc1, fc3, fc2, bemap, nbclamp, valid):
    p, h = xs.shape
    _, _, inter = fc1.shape
    n_nb = p // _BLK
    n_it = inter // _IB

    def wmap(nb, it, bm, nc, vl):
        return (bm[nb], jnp.where(vl[nb] == 1, it, n_it - 1))

    def map13(nb, it, bm, nc, vl):
        be_i, it_i = wmap(nb, it, bm, nc, vl)
        return (be_i, 0, it_i)

    def map2(nb, it, bm, nc, vl):
        be_i, it_i = wmap(nb, it, bm, nc, vl)
        return (be_i, it_i, 0)

    return pl.pallas_call(
        functools.partial(_ffn_body, n_it=n_it),
        grid_spec=pltpu.PrefetchScalarGridSpec(
            num_scalar_prefetch=3,
            grid=(n_nb, n_it),
            in_specs=[
                pl.BlockSpec((_BLK, h), lambda nb, it, bm, nc, vl: (nc[nb], 0)),
                pl.BlockSpec((1, h, _IB), map13),
                pl.BlockSpec((1, h, _IB), map13),
                pl.BlockSpec((1, _IB, h), map2),
            ],
            out_specs=pl.BlockSpec((_BLK, h), lambda nb, it, bm, nc, vl: (nc[nb], 0)),
            scratch_shapes=[pltpu.VMEM((_BLK, h), jnp.float32)],
        ),
        out_shape=jax.ShapeDtypeStruct((p, h), jnp.float32),
        compiler_params=pltpu.CompilerParams(
            dimension_semantics=("arbitrary", "arbitrary"),
        ),
    )(bemap, nbclamp, valid, xs, fc1, fc3, fc2)


def _vgather(vec, idx):
    return lax.gather(
        vec, idx[:, None],
        lax.GatherDimensionNumbers(offset_dims=(), collapsed_slice_dims=(0,),
                                   start_index_map=(0,)),
        (1,), mode=lax.GatherScatterMode.PROMISE_IN_BOUNDS)


def _vsplat(vec, i):
    return _vgather(vec, jnp.full((_NL,), i, dtype=jnp.int32))


def _scatter_sc(x, pos1, pos2, n, h, ptotal):
    nw = _NC * _NS
    tpw = n // nw
    mesh = plsc.VectorSubcoreMesh(core_axis_name="c", subcore_axis_name="s")

    @functools.partial(
        pl.kernel, mesh=mesh,
        out_type=jax.ShapeDtypeStruct((ptotal, h), jnp.float32),
        scratch_types=[
            pltpu.VMEM((tpw, h), jnp.float32),
            pltpu.VMEM((2, tpw), jnp.int32),
            pltpu.SemaphoreType.DMA,
        ],
    )
    def body(x_hbm, pos1_hbm, pos2_hbm, xs_hbm, xrows_v, idx2_v, sem):
        wid = lax.axis_index("s") * _NC + lax.axis_index("c")
        base = pl.ds(wid * tpw, tpw)
        pltpu.sync_copy(pos1_hbm.at[base], idx2_v.at[0])
        pltpu.sync_copy(pos2_hbm.at[base], idx2_v.at[1])
        pltpu.sync_copy(x_hbm.at[base], xrows_v)
        pltpu.async_copy(xrows_v, xs_hbm.at[idx2_v.at[0]], sem).wait()
        pltpu.async_copy(xrows_v, xs_hbm.at[idx2_v.at[1]], sem).wait()

    return body(x, pos1, pos2)


def _combine_sc(ys, posr, wr, n, h):
    nw = _NC * _NS
    tpw = n // nw
    nch = tpw // _NL
    mesh = plsc.VectorSubcoreMesh(core_axis_name="c", subcore_axis_name="s")

    @functools.partial(
        pl.kernel, mesh=mesh,
        out_type=jax.ShapeDtypeStruct((n, h), jnp.float32),
        scratch_types=[
            pltpu.VMEM((nch, 2 * _NL), jnp.int32),
            pltpu.VMEM((nch * 2 * _NL + _NL,), jnp.float32),
            pltpu.VMEM((2 * _NL, h), jnp.float32),
            pltpu.VMEM((_NL, h), jnp.float32),
            pltpu.SemaphoreType.DMA,
        ],
    )
    def body(ys_hbm, pos_hbm, w_hbm, out_hbm, idx_v, w_v, rows_v, out_v, sem):
        wid = lax.axis_index("s") * _NC + lax.axis_index("c")
        pltpu.sync_copy(pos_hbm.at[pl.ds(wid * nch, nch)], idx_v)
        pltpu.sync_copy(w_hbm.at[pl.ds(wid * nch * 2 * _NL, nch * 2 * _NL)],
                        w_v.at[pl.ds(0, nch * 2 * _NL)])
        for c in range(nch):
            pltpu.async_copy(ys_hbm.at[idx_v.at[c]], rows_v, sem).wait()

            def tok(t, _):
                wpair = w_v[pl.ds(c * 2 * _NL + 2 * t, _NL)]
                w1s = jnp.full((_NL,), wpair[0])
                w2s = jnp.full((_NL,), wpair[1])
                for sl in range(h // _NL):
                    d = pl.ds(sl * _NL, _NL)
                    out_v[t, d] = (w1s * rows_v[2 * t, d]
                                   + w2s * rows_v[2 * t + 1, d])
                return 0

            lax.fori_loop(0, _NL, tok, 0)
            pltpu.sync_copy(out_v, out_hbm.at[pl.ds(wid * tpw + c * _NL, _NL)])

    return body(ys, posr, wr)


def kernel(x, router_probs, fc1, fc2, fc3):
    n, h = x.shape
    e = fc1.shape[0]
    nk = n * _TOPK

    eids = jnp.arange(e, dtype=jnp.int32)
    a1 = jnp.argmax(router_probs, axis=1).astype(jnp.int32)
    m1 = jnp.max(router_probs, axis=1)
    rp2 = jnp.where(eids[None, :] == a1[:, None], -jnp.inf, router_probs)
    a2 = jnp.argmax(rp2, axis=1).astype(jnp.int32)
    m2 = jnp.max(rp2, axis=1)
    s = m1 + m2
    topk_probs = jnp.stack([m1 / s, m2 / s], axis=1)
    flat_e = jnp.stack([a1, a2], axis=1).reshape(-1)

    oh = (flat_e[:, None] == eids[None, :]).astype(jnp.int32)
    nch = 32
    ohf = oh.astype(jnp.float32).reshape(nch, nk // nch, e)
    r1 = jnp.arange(nk // nch)
    t_in = (r1[:, None] >= r1[None, :]).astype(jnp.float32)
    s1 = jnp.einsum('ij,cje->cie', t_in, ohf,
                    preferred_element_type=jnp.float32)
    ctot = jnp.sum(ohf, axis=1)
    r2 = jnp.arange(nch)
    t_ex = (r2[:, None] > r2[None, :]).astype(jnp.float32)
    s2 = jnp.dot(t_ex, ctot, preferred_element_type=jnp.float32)
    csum = (s1 + s2[:, None, :]).reshape(nk, e).astype(jnp.int32)
    counts = jnp.sum(ctot, axis=0).astype(jnp.int32)
    rank = jnp.sum(csum * oh, axis=1) - 1

    padded = ((counts + _BLK - 1) // _BLK) * _BLK
    pstart = jnp.concatenate(
        [jnp.zeros((1,), jnp.int32), jnp.cumsum(padded)[:-1].astype(jnp.int32)])
    pos = pstart[flat_e] + rank

    ptotal = -(-(nk + e * _BLK) // _BLK) * _BLK
    n_nb = ptotal // _BLK

    ends = jnp.cumsum(padded).astype(jnp.int32)
    blk_starts = jnp.arange(n_nb, dtype=jnp.int32) * _BLK
    be = jnp.sum(blk_starts[:, None] >= ends[None, :], axis=1).astype(jnp.int32)
    valid = (be < e).astype(jnp.int32)
    last_valid = jnp.maximum(jnp.sum(valid) - 1, 0)
    nbclamp = jnp.minimum(jnp.arange(n_nb, dtype=jnp.int32), last_valid)
    bemap = jnp.minimum(be, be[last_valid])

    pos2d = pos.reshape(n, _TOPK)
    xs = _scatter_sc(x, pos2d[:, 0], pos2d[:, 1], n, h, ptotal)
    ys = _grouped_ffn(xs, fc1, fc3, fc2, bemap, nbclamp, valid)
    posr = pos.reshape(nk // (2 * _NL), 2 * _NL)
    wflat = topk_probs.reshape(-1)
    return _combine_sc(ys, posr, wflat, n, h)


# speedup vs baseline: 1.1449x; 1.1185x over previous
import functools

import jax
import jax.numpy as jnp
from jax import lax
from jax.experimental import pallas as pl
from jax.experimental.pallas import tpu as pltpu
from jax.experimental.pallas import tpu_sc as plsc

_NC, _NS, _NL = 2, 16, 16

_TOPK = 2
_BLK = 576
_IB = 1024


def _ffn_body(meta_ref, xs_ref, fc1_ref, fc3_ref, fc2_ref, out_ref, acc_ref,
              *, n_it):
    nb = pl.program_id(0)
    it = pl.program_id(1)

    @pl.when(meta_ref[2 * _NL + nb] == 1)
    def _():
        @pl.when(it == 0)
        def _():
            acc_ref[...] = jnp.zeros_like(acc_ref)

        xs = xs_ref[...].astype(jnp.bfloat16)
        h1 = jnp.dot(xs, fc1_ref[0].astype(jnp.bfloat16),
                     preferred_element_type=jnp.float32)
        h3 = jnp.dot(xs, fc3_ref[0].astype(jnp.bfloat16),
                     preferred_element_type=jnp.float32)
        act = h1 * jax.nn.sigmoid(h1) * h3
        acc_ref[...] += jnp.dot(act.astype(jnp.bfloat16),
                                fc2_ref[0].astype(jnp.bfloat16),
                                preferred_element_type=jnp.float32)

        @pl.when(it == n_it - 1)
        def _():
            out_ref[...] = acc_ref[...]


def _grouped_ffn(xs, fc1, fc3, fc2, meta):
    p, h = xs.shape
    _, _, inter = fc1.shape
    n_nb = p // _BLK
    n_it = inter // _IB

    def wmap(nb, it, mt):
        return (mt[nb], jnp.where(mt[2 * _NL + nb] == 1, it, n_it - 1))

    def map13(nb, it, mt):
        be_i, it_i = wmap(nb, it, mt)
        return (be_i, 0, it_i)

    def map2(nb, it, mt):
        be_i, it_i = wmap(nb, it, mt)
        return (be_i, it_i, 0)

    return pl.pallas_call(
        functools.partial(_ffn_body, n_it=n_it),
        grid_spec=pltpu.PrefetchScalarGridSpec(
            num_scalar_prefetch=1,
            grid=(n_nb, n_it),
            in_specs=[
                pl.BlockSpec((_BLK, h), lambda nb, it, mt: (mt[_NL + nb], 0)),
                pl.BlockSpec((1, h, _IB), map13),
                pl.BlockSpec((1, h, _IB), map13),
                pl.BlockSpec((1, _IB, h), map2),
            ],
            out_specs=pl.BlockSpec((_BLK, h), lambda nb, it, mt: (mt[_NL + nb], 0)),
            scratch_shapes=[pltpu.VMEM((_BLK, h), jnp.float32)],
        ),
        out_shape=jax.ShapeDtypeStruct((p, h), jnp.float32),
        compiler_params=pltpu.CompilerParams(
            dimension_semantics=("arbitrary", "arbitrary"),
        ),
    )(meta, xs, fc1, fc3, fc2)


def _route2_sc(x, e1, e2, rank1, rank2, pstart16, n, h, e, ptotal):
    nw = _NC * _NS
    tpw = n // nw
    ngr = tpw // _NL
    mesh = plsc.VectorSubcoreMesh(core_axis_name="c", subcore_axis_name="s")

    @functools.partial(
        pl.kernel, mesh=mesh,
        out_type=[
            jax.ShapeDtypeStruct((ptotal, h), jnp.float32),
            jax.ShapeDtypeStruct((n,), jnp.int32),
            jax.ShapeDtypeStruct((n,), jnp.int32),
        ],
        scratch_types=[
            pltpu.VMEM((tpw, h), jnp.float32),
            pltpu.VMEM((2, tpw), jnp.int32),
            pltpu.VMEM((tpw,), jnp.int32),
            pltpu.VMEM((tpw,), jnp.int32),
            pltpu.VMEM((tpw,), jnp.int32),
            pltpu.VMEM((tpw,), jnp.int32),
            pltpu.VMEM((_NL,), jnp.int32),
            pltpu.SemaphoreType.DMA,
        ],
    )
    def body(x_hbm, e1_hbm, e2_hbm, r1_hbm, r2_hbm, ps_hbm, xs_hbm,
             pos1_hbm, pos2_hbm, xrows_v, idx2_v, e1_v, e2_v,
             r1_v, r2_v, ps_v, sem):
        wid = lax.axis_index("s") * _NC + lax.axis_index("c")
        lanes = lax.iota(jnp.int32, _NL)
        base = pl.ds(wid * tpw, tpw)
        pltpu.sync_copy(x_hbm.at[base], xrows_v)
        pltpu.sync_copy(e1_hbm.at[base], e1_v)
        pltpu.sync_copy(e2_hbm.at[base], e2_v)
        pltpu.sync_copy(r1_hbm.at[base], r1_v)
        pltpu.sync_copy(r2_hbm.at[base], r2_v)
        pltpu.sync_copy(ps_hbm, ps_v)
        psv = ps_v[pl.ds(0, _NL)]

        for g in range(ngr):
            d = pl.ds(g * _NL, _NL)
            ev1 = e1_v[d]
            ev2 = e2_v[d]
            ps1 = jnp.zeros((_NL,), jnp.int32)
            ps2 = jnp.zeros((_NL,), jnp.int32)
            for ex in range(e):
                pe = jnp.full((_NL,), psv[ex])
                ps1 = jnp.where(ev1 == ex, pe, ps1)
                ps2 = jnp.where(ev2 == ex, pe, ps2)
            idx2_v[0, d] = ps1 + r1_v[d]
            idx2_v[1, d] = ps2 + r2_v[d]

        pltpu.sync_copy(idx2_v.at[0], pos1_hbm.at[base])
        pltpu.sync_copy(idx2_v.at[1], pos2_hbm.at[base])
        pltpu.async_copy(xrows_v, xs_hbm.at[idx2_v.at[0]], sem).wait()
        pltpu.async_copy(xrows_v, xs_hbm.at[idx2_v.at[1]], sem).wait()


    return body(x, e1, e2, rank1, rank2, pstart16)


def _combine_sc(ys, pos1, pos2, w1, w2, n, h):
    nw = _NC * _NS
    tpw = n // nw
    nch = tpw // _NL
    mesh = plsc.VectorSubcoreMesh(core_axis_name="c", subcore_axis_name="s")

    @functools.partial(
        pl.kernel, mesh=mesh,
        out_type=jax.ShapeDtypeStruct((n, h), jnp.float32),
        scratch_types=[
            pltpu.VMEM((tpw,), jnp.int32),
            pltpu.VMEM((tpw,), jnp.int32),
            pltpu.VMEM((tpw + _NL,), jnp.float32),
            pltpu.VMEM((tpw + _NL,), jnp.float32),
            pltpu.VMEM((_NL, h), jnp.float32),
            pltpu.VMEM((_NL, h), jnp.float32),
            pltpu.VMEM((_NL, h), jnp.float32),
            pltpu.SemaphoreType.DMA,
        ],
    )
    def body(ys_hbm, pos1_hbm, pos2_hbm, w1_hbm, w2_hbm, out_hbm,
             idxa_v, idxb_v, wa_v, wb_v, rowsa_v, rowsb_v, out_v, sem):
        wid = lax.axis_index("s") * _NC + lax.axis_index("c")
        base = pl.ds(wid * tpw, tpw)
        pltpu.sync_copy(pos1_hbm.at[base], idxa_v)
        pltpu.sync_copy(pos2_hbm.at[base], idxb_v)
        pltpu.sync_copy(w1_hbm.at[base], wa_v.at[pl.ds(0, tpw)])
        pltpu.sync_copy(w2_hbm.at[base], wb_v.at[pl.ds(0, tpw)])
        for c in range(nch):
            ca = pltpu.async_copy(
                ys_hbm.at[idxa_v.at[pl.ds(c * _NL, _NL)]], rowsa_v, sem)
            cb = pltpu.async_copy(
                ys_hbm.at[idxb_v.at[pl.ds(c * _NL, _NL)]], rowsb_v, sem)
            ca.wait()
            cb.wait()

            def tok(t, _):
                wa = wa_v[pl.ds(c * _NL + t, _NL)]
                wb = wb_v[pl.ds(c * _NL + t, _NL)]
                w1s = jnp.full((_NL,), wa[0])
                w2s = jnp.full((_NL,), wb[0])
                for sl in range(h // _NL):
                    d = pl.ds(sl * _NL, _NL)
                    out_v[t, d] = w1s * rowsa_v[t, d] + w2s * rowsb_v[t, d]
                return 0

            lax.fori_loop(0, _NL, tok, 0)
            pltpu.sync_copy(out_v, out_hbm.at[pl.ds(wid * tpw + c * _NL, _NL)])

    return body(ys, pos1, pos2, w1, w2)


def kernel(x, router_probs, fc1, fc2, fc3):
    n, h = x.shape
    e = fc1.shape[0]
    nk = n * _TOPK
    ptotal = -(-(nk + e * _BLK) // _BLK) * _BLK

    eids = jnp.arange(e, dtype=jnp.int32)
    a1 = jnp.argmax(router_probs, axis=1).astype(jnp.int32)
    m1 = jnp.max(router_probs, axis=1)
    rp2 = jnp.where(eids[None, :] == a1[:, None], -jnp.inf, router_probs)
    a2 = jnp.argmax(rp2, axis=1).astype(jnp.int32)
    m2 = jnp.max(rp2, axis=1)
    s = m1 + m2
    w1 = m1 / s
    w2 = m2 / s
    flat_e = jnp.stack([a1, a2], axis=1).reshape(-1)

    oh = (flat_e[:, None] == eids[None, :]).astype(jnp.int32)
    nch = 32
    ohf = oh.astype(jnp.float32).reshape(nch, nk // nch, e)
    r1 = jnp.arange(nk // nch)
    t_in = (r1[:, None] >= r1[None, :]).astype(jnp.float32)
    s1 = jnp.einsum('ij,cje->cie', t_in, ohf,
                    preferred_element_type=jnp.float32)
    ctot = jnp.sum(ohf, axis=1)
    r2 = jnp.arange(nch)
    t_ex = (r2[:, None] > r2[None, :]).astype(jnp.float32)
    s2 = jnp.dot(t_ex, ctot, preferred_element_type=jnp.float32)
    csum = (s1 + s2[:, None, :]).reshape(nk, e).astype(jnp.int32)
    counts = jnp.sum(ctot, axis=0).astype(jnp.int32)
    rank = (jnp.sum(csum * oh, axis=1) - 1).reshape(n, _TOPK)

    n_nb = ptotal // _BLK
    padded = ((counts + _BLK - 1) // _BLK) * _BLK
    ends = jnp.cumsum(padded).astype(jnp.int32)
    pstart16 = jnp.zeros((_NL,), jnp.int32).at[1:e].set(ends[:-1])
    blk_starts = jnp.arange(n_nb, dtype=jnp.int32) * _BLK
    be = jnp.sum(blk_starts[:, None] >= ends[None, :], axis=1).astype(jnp.int32)
    valid = (be < e).astype(jnp.int32)
    last_valid = jnp.maximum(jnp.sum(valid) - 1, 0)
    nbclamp = jnp.minimum(jnp.arange(n_nb, dtype=jnp.int32), last_valid)
    bemap = jnp.minimum(be, be[last_valid])
    meta = jnp.concatenate([bemap, nbclamp, valid])

    xs, pos1, pos2 = _route2_sc(
        x, a1, a2, rank[:, 0], rank[:, 1], pstart16, n, h, e, ptotal)
    ys = _grouped_ffn(xs, fc1, fc3, fc2, meta)
    return _combine_sc(ys, pos1, pos2, w1, w2, n, h)
